# baseline (speedup 1.0000x reference)
import jax
import jax.numpy as jnp
from jax.experimental import pallas as pl

N = 207
NP = 208
HID = 64
T = 12
D_IN = 2
NCLS = 5
BS = 4


def _cheb(S, Yall, O):
    Y0 = Yall[:, :O]
    Y1 = Yall[:, O:2 * O]
    Y2 = Yall[:, 2 * O:]
    U = S @ Y2
    Z = S @ (Y1 + 2.0 * U)
    return Y0 - Y2 + Z


def _dcrnn_kernel(inp_ref, seq_ref, s_ref,
                  wg0i_ref, wg0s_ref, bg0_ref, wc0i_ref, wc0s_ref, bc0_ref,
                  wi1_ref, wg1s_ref, bg1_ref, wc1s_ref, bc1_ref,
                  wfc_ref, bfc_ref, out_ref):
    S = s_ref[...]
    Wg0i = wg0i_ref[...]
    Wg0s = wg0s_ref[...]
    Wc0i = wc0i_ref[...]
    Wc0s = wc0s_ref[...]
    Wi1 = wi1_ref[...]
    Wg1s = wg1s_ref[...]
    Wc1s = wc1s_ref[...]
    bg0 = bg0_ref[...]
    bc0 = bc0_ref[...]
    bg1 = bg1_ref[...]
    bc1 = bc1_ref[...]

    st0 = [jnp.zeros((NP, HID), jnp.float32) for _ in range(BS)]
    st1 = [jnp.zeros((NP, HID), jnp.float32) for _ in range(BS)]
    last = [jnp.zeros((NP, HID), jnp.float32) for _ in range(BS)]

    for t in range(T):
        for b in range(BS):
            xt = inp_ref[b, t]
            st = st0[b]
            Yg = xt @ Wg0i + st @ Wg0s
            val = jax.nn.sigmoid(_cheb(S, Yg, 2 * HID) + bg0)
            r = val[:, :HID]
            u = val[:, HID:]
            Yc = xt @ Wc0i + (r * st) @ Wc0s
            c = jnp.tanh(_cheb(S, Yc, HID) + bc0)
            st0[b] = u * st + (1.0 - u) * c
            P = st0[b] @ Wi1
            st = st1[b]
            Yg = P[:, :6 * HID] + st @ Wg1s
            val = jax.nn.sigmoid(_cheb(S, Yg, 2 * HID) + bg1)
            r = val[:, :HID]
            u = val[:, HID:]
            Yc = P[:, 6 * HID:] + (r * st) @ Wc1s
            c = jnp.tanh(_cheb(S, Yc, HID) + bc1)
            st1[b] = u * st + (1.0 - u) * c
            L = seq_ref[b, 0, 0]
            last[b] = jnp.where(L == t + 1, st1[b], last[b])

    node = jax.lax.broadcasted_iota(jnp.int32, (NP, NCLS), 0)
    for b in range(BS):
        logits = jax.nn.relu(last[b]) @ wfc_ref[...] + bfc_ref[...]
        logits = jnp.where(node < N, logits, -jnp.inf)
        out_ref[b, 0, :] = jnp.max(logits, axis=0)


def _deint(W, d_in):
    isz = W.shape[0] // 3
    O = W.shape[1]
    Wm = jnp.transpose(W.reshape(isz, 3, O), (1, 0, 2))
    Wcat = jnp.concatenate([Wm[0], Wm[1], Wm[2]], axis=1)
    return Wcat[:d_in], Wcat[d_in:]


@jax.jit
def kernel(input_seq, seq_lengths, supports, Wg0, bg0, Wc0, bc0,
           Wg1, bg1, Wc1, bc1, Wfc, bfc):
    B = input_seq.shape[0]
    S = jnp.pad(supports[0], ((0, NP - N), (0, NP - N)))
    inp = jnp.pad(input_seq, ((0, 0), (0, 0), (0, NP - N), (0, 0)))
    Wg0i, Wg0s = _deint(Wg0, D_IN)
    Wc0i, Wc0s = _deint(Wc0, D_IN)
    Wg1i, Wg1s = _deint(Wg1, HID)
    Wc1i, Wc1s = _deint(Wc1, HID)
    Wi1 = jnp.concatenate([Wg1i, Wc1i], axis=1)
    seq = seq_lengths.astype(jnp.int32).reshape(B, 1, 1)

    def c(shape):
        return pl.BlockSpec(shape, lambda g: (0,) * len(shape))

    grid_spec = pl.GridSpec(
        grid=(B // BS,),
        in_specs=[
            pl.BlockSpec((BS, T, NP, D_IN), lambda g: (g, 0, 0, 0)),
            pl.BlockSpec((BS, 1, 1), lambda g: (g, 0, 0)),
            c((NP, NP)),
            c(Wg0i.shape), c(Wg0s.shape), c((1, 2 * HID)),
            c(Wc0i.shape), c(Wc0s.shape), c((1, HID)),
            c(Wi1.shape), c(Wg1s.shape), c((1, 2 * HID)),
            c(Wc1s.shape), c((1, HID)),
            c((HID, NCLS)), c((1, NCLS)),
        ],
        out_specs=pl.BlockSpec((BS, 1, NCLS), lambda g: (g, 0, 0)),
    )
    out = pl.pallas_call(
        _dcrnn_kernel,
        grid_spec=grid_spec,
        out_shape=jax.ShapeDtypeStruct((B, 1, NCLS), jnp.float32),
    )(inp, seq, S,
      Wg0i, Wg0s, bg0.reshape(1, -1), Wc0i, Wc0s, bc0.reshape(1, -1),
      Wi1, Wg1s, bg1.reshape(1, -1), Wc1s, bc1.reshape(1, -1),
      Wfc, bfc.reshape(1, -1))
    return out.reshape(B, NCLS)


# speedup vs baseline: 1.1371x; 1.1371x over previous
import jax
import jax.numpy as jnp
from jax.experimental import pallas as pl

N = 207
HID = 64
T = 12
D_IN = 2
NCLS = 5
BS = 4


def _gconv(S, inp, st, Wmi, Wms):
    Y0 = inp @ Wmi[0] + st @ Wms[0]
    Y1 = inp @ Wmi[1] + st @ Wms[1]
    Y2 = inp @ Wmi[2] + st @ Wms[2]
    U = S @ Y2
    Z = S @ (Y1 + 2.0 * U)
    return Y0 - Y2 + Z


def _cell(S, inp, st, Wgi, Wgs, bg, Wci, Wcs, bc):
    val = jax.nn.sigmoid(_gconv(S, inp, st, Wgi, Wgs) + bg)
    r = val[:, :HID]
    u = val[:, HID:]
    c = jnp.tanh(_gconv(S, inp, r * st, Wci, Wcs) + bc)
    return u * st + (1.0 - u) * c


def _dcrnn_kernel(inp_ref, seq_ref, s_ref,
                  wg0i_ref, wg0s_ref, bg0_ref, wc0i_ref, wc0s_ref, bc0_ref,
                  wg1i_ref, wg1s_ref, bg1_ref, wc1i_ref, wc1s_ref, bc1_ref,
                  wfc_ref, bfc_ref, out_ref):
    S = s_ref[...]
    Wg0i = [wg0i_ref[m] for m in range(3)]
    Wg0s = [wg0s_ref[m] for m in range(3)]
    Wc0i = [wc0i_ref[m] for m in range(3)]
    Wc0s = [wc0s_ref[m] for m in range(3)]
    Wg1i = [wg1i_ref[m] for m in range(3)]
    Wg1s = [wg1s_ref[m] for m in range(3)]
    Wc1i = [wc1i_ref[m] for m in range(3)]
    Wc1s = [wc1s_ref[m] for m in range(3)]
    bg0 = bg0_ref[...]
    bc0 = bc0_ref[...]
    bg1 = bg1_ref[...]
    bc1 = bc1_ref[...]

    st0 = [jnp.zeros((N, HID), jnp.float32) for _ in range(BS)]
    st1 = [jnp.zeros((N, HID), jnp.float32) for _ in range(BS)]
    last = [jnp.zeros((N, HID), jnp.float32) for _ in range(BS)]

    for t in range(T):
        for b in range(BS):
            xt = inp_ref[b, t]
            st0[b] = _cell(S, xt, st0[b], Wg0i, Wg0s, bg0, Wc0i, Wc0s, bc0)
            st1[b] = _cell(S, st0[b], st1[b], Wg1i, Wg1s, bg1, Wc1i, Wc1s, bc1)
            L = seq_ref[b, 0, 0]
            last[b] = jnp.where(L == t + 1, st1[b], last[b])

    for b in range(BS):
        logits = jax.nn.relu(last[b]) @ wfc_ref[...] + bfc_ref[...]
        out_ref[b, 0, :] = jnp.max(logits, axis=0)


def _split_w(W, d_in):
    isz = W.shape[0] // 3
    O = W.shape[1]
    Wm = jnp.transpose(W.reshape(isz, 3, O), (1, 0, 2))
    return Wm[:, :d_in, :], Wm[:, d_in:, :]


@jax.jit
def kernel(input_seq, seq_lengths, supports, Wg0, bg0, Wc0, bc0,
           Wg1, bg1, Wc1, bc1, Wfc, bfc):
    B = input_seq.shape[0]
    S = supports[0]
    Wg0i, Wg0s = _split_w(Wg0, D_IN)
    Wc0i, Wc0s = _split_w(Wc0, D_IN)
    Wg1i, Wg1s = _split_w(Wg1, HID)
    Wc1i, Wc1s = _split_w(Wc1, HID)
    seq = seq_lengths.astype(jnp.int32).reshape(B, 1, 1)

    def c(shape):
        return pl.BlockSpec(shape, lambda g: (0,) * len(shape))

    grid_spec = pl.GridSpec(
        grid=(B // BS,),
        in_specs=[
            pl.BlockSpec((BS, T, N, D_IN), lambda g: (g, 0, 0, 0)),
            pl.BlockSpec((BS, 1, 1), lambda g: (g, 0, 0)),
            c((N, N)),
            c(Wg0i.shape), c(Wg0s.shape), c((1, 2 * HID)),
            c(Wc0i.shape), c(Wc0s.shape), c((1, HID)),
            c(Wg1i.shape), c(Wg1s.shape), c((1, 2 * HID)),
            c(Wc1i.shape), c(Wc1s.shape), c((1, HID)),
            c((HID, NCLS)), c((1, NCLS)),
        ],
        out_specs=pl.BlockSpec((BS, 1, NCLS), lambda g: (g, 0, 0)),
    )
    out = pl.pallas_call(
        _dcrnn_kernel,
        grid_spec=grid_spec,
        out_shape=jax.ShapeDtypeStruct((B, 1, NCLS), jnp.float32),
    )(input_seq, seq, S,
      Wg0i, Wg0s, bg0.reshape(1, -1), Wc0i, Wc0s, bc0.reshape(1, -1),
      Wg1i, Wg1s, bg1.reshape(1, -1), Wc1i, Wc1s, bc1.reshape(1, -1),
      Wfc, bfc.reshape(1, -1))
    return out.reshape(B, NCLS)
